# baseline (speedup 1.0000x reference)
import functools

import jax
import jax.numpy as jnp
from jax import lax
from jax.experimental import pallas as pl
from jax.experimental.pallas import tpu as pltpu
from jax.experimental.pallas import tpu_sc as plsc

_B, _N = 8, 8192
_M1, _K1 = 512, 32
_M2, _K2 = 128, 16
_NW = 32


def _fps_kernel(x_ref, y_ref, z_ref, ox_ref, oy_ref, oz_ref):
    x = x_ref[...]
    y = y_ref[...]
    z = z_ref[...]
    lane_n = lax.broadcasted_iota(jnp.int32, (_B, _N), 1)
    lane_m = lax.broadcasted_iota(jnp.int32, (_B, _M1), 1)

    cx0 = x[:, 0:1]
    cy0 = y[:, 0:1]
    cz0 = z[:, 0:1]
    ox0 = jnp.where(lane_m == 0, cx0, 0.0)
    oy0 = jnp.where(lane_m == 0, cy0, 0.0)
    oz0 = jnp.where(lane_m == 0, cz0, 0.0)
    dmin0 = jnp.full((_B, _N), jnp.inf, dtype=jnp.float32)

    def body(i, carry):
        cx, cy, cz, ox, oy, oz, dmin = carry
        dx = x - cx
        dy = y - cy
        dz = z - cz
        d = dx * dx + dy * dy + dz * dz
        dmin = jnp.minimum(dmin, d)
        m = jnp.max(dmin, axis=1, keepdims=True)
        cand = jnp.where(dmin == m, lane_n, _N)
        nxt = jnp.min(cand, axis=1, keepdims=True)
        sel = lane_n == nxt
        ncx = jnp.sum(jnp.where(sel, x, 0.0), axis=1, keepdims=True)
        ncy = jnp.sum(jnp.where(sel, y, 0.0), axis=1, keepdims=True)
        ncz = jnp.sum(jnp.where(sel, z, 0.0), axis=1, keepdims=True)
        hit = lane_m == i
        ox = jnp.where(hit, ncx, ox)
        oy = jnp.where(hit, ncy, oy)
        oz = jnp.where(hit, ncz, oz)
        return ncx, ncy, ncz, ox, oy, oz, dmin

    _, _, _, ox, oy, oz, _ = lax.fori_loop(
        1, _M1, body, (cx0, cy0, cz0, ox0, oy0, oz0, dmin0))
    ox_ref[...] = ox
    oy_ref[...] = oy
    oz_ref[...] = oz


def _fps(ct):
    return pl.pallas_call(
        _fps_kernel,
        out_shape=[jax.ShapeDtypeStruct((_B, _M1), jnp.float32)] * 3,
    )(ct[0], ct[1], ct[2])


def _knn_kernel(ct_ref, cen_ref, kidx_ref, d_ref, *, m, n, k):
    ct = ct_ref[0]
    cen = cen_ref[0]
    xx = jnp.sum(ct * ct, axis=0, keepdims=True)
    cc = jnp.sum(cen * cen, axis=1, keepdims=True)
    e = jnp.dot(cen, ct, preferred_element_type=jnp.float32)
    d_ref[...] = cc + xx - 2.0 * e
    lane_n = lax.broadcasted_iota(jnp.int32, (m, n), 1)
    lane_k = lax.broadcasted_iota(jnp.int32, (m, k), 1)

    def step(j, carry):
        prev, kidx = carry
        dm = jnp.where(lane_n == prev, jnp.inf, d_ref[...])
        d_ref[...] = dm
        nxt = jnp.argmin(dm, axis=1).astype(jnp.int32)[:, None]
        kidx = jnp.where(lane_k == j, nxt, kidx)
        return nxt, kidx

    _, kidx = lax.fori_loop(
        0, k, step,
        (jnp.full((m, 1), -1, jnp.int32), jnp.zeros((m, k), jnp.int32)))
    kidx_ref[0] = kidx


def _knn(ct8, cen, m, n, k):
    return pl.pallas_call(
        functools.partial(_knn_kernel, m=m, n=n, k=k),
        grid=(_B,),
        in_specs=[
            pl.BlockSpec((1, 8, n), lambda b: (b, 0, 0)),
            pl.BlockSpec((1, m, 8), lambda b: (b, 0, 0)),
        ],
        out_specs=pl.BlockSpec((1, m, k), lambda b: (b, 0, 0)),
        out_shape=jax.ShapeDtypeStruct((_B, m, k), jnp.int32),
        scratch_shapes=[pltpu.VMEM((m, n), jnp.float32)],
    )(ct8, cen)


def _sc_gather(table, idx):
    ni = idx.shape[0]
    d = table.shape[1]
    bpw = ni // _NW
    mesh = plsc.VectorSubcoreMesh(core_axis_name="c", subcore_axis_name="s")

    @functools.partial(
        pl.kernel,
        out_type=jax.ShapeDtypeStruct((ni, d), jnp.float32),
        mesh=mesh,
        scratch_types=[
            pltpu.VMEM((bpw,), jnp.int32),
            pltpu.VMEM((bpw, d), jnp.float32),
            pltpu.SemaphoreType.DMA,
        ],
        compiler_params=pltpu.CompilerParams(use_tc_tiling_on_sc=False),
    )
    def gk(table_hbm, idx_hbm, out_hbm, idx_v, rows_v, sem):
        wid = lax.axis_index("s") * 2 + lax.axis_index("c")
        base = wid * bpw
        pltpu.sync_copy(idx_hbm.at[pl.ds(base, bpw)], idx_v)
        pltpu.async_copy(table_hbm.at[idx_v], rows_v, sem).wait()
        pltpu.sync_copy(rows_v, out_hbm.at[pl.ds(base, bpw)])

    return gk(table, idx)


def _enc_kernel(p_ref, c_ref, w1_ref, b1_ref, w2_ref, b2_ref, w3_ref, b3_ref,
                o_ref, *, group):
    g = p_ref[...]
    c = c_ref[...]
    r, kd = g.shape
    cb = jnp.broadcast_to(c[:, None, :], (r // group, group, kd)).reshape(r, kd)
    h = g - cb
    h = jnp.dot(h, w1_ref[...], preferred_element_type=jnp.float32)
    h = jnp.maximum(h + b1_ref[...], 0.0)
    h = jnp.dot(h, w2_ref[...], preferred_element_type=jnp.float32)
    h = jnp.maximum(h + b2_ref[...], 0.0)
    h = jnp.dot(h, w3_ref[...], preferred_element_type=jnp.float32)
    h = jnp.maximum(h + b3_ref[...], 0.0)
    o_ref[...] = jnp.max(h.reshape(r // group, group, -1), axis=1)


def _encode(p, cen, w1, b1, w2, b2, w3, b3, group, tile_rows):
    rows, kdim = p.shape
    grid = rows // tile_rows
    otile = tile_rows // group
    c1, c2, c3 = w1.shape[1], w2.shape[1], w3.shape[1]
    return pl.pallas_call(
        functools.partial(_enc_kernel, group=group),
        grid=(grid,),
        in_specs=[
            pl.BlockSpec((tile_rows, kdim), lambda i: (i, 0)),
            pl.BlockSpec((otile, kdim), lambda i: (i, 0)),
            pl.BlockSpec((kdim, c1), lambda i: (0, 0)),
            pl.BlockSpec((1, c1), lambda i: (0, 0)),
            pl.BlockSpec((c1, c2), lambda i: (0, 0)),
            pl.BlockSpec((1, c2), lambda i: (0, 0)),
            pl.BlockSpec((c2, c3), lambda i: (0, 0)),
            pl.BlockSpec((1, c3), lambda i: (0, 0)),
        ],
        out_specs=pl.BlockSpec((otile, c3), lambda i: (i, 0)),
        out_shape=jax.ShapeDtypeStruct((rows // group, c3), jnp.float32),
    )(p, cen, w1, b1, w2, b2, w3, b3)


def kernel(coords, features, W1a, b1a, W2a, b2a, W3a, b3a,
           W1b, b1b, W2b, b2b, W3b, b3b):
    ct = jnp.transpose(coords, (2, 0, 1))
    ox, oy, oz = _fps(ct)
    centers1 = jnp.stack([ox, oy, oz], axis=-1)
    zm = jnp.zeros((_B, _M1), jnp.float32)
    cen_pad = jnp.stack([ox, oy, oz, zm, zm, zm, zm, zm], axis=-1)
    ct8 = jnp.concatenate(
        [ct, jnp.zeros((5, _B, _N), jnp.float32)], axis=0
    ).transpose(1, 0, 2)

    kidx = _knn(ct8, cen_pad, _M1, _N, _K1)
    gidx = kidx + (jnp.arange(_B, dtype=jnp.int32) * _N)[:, None, None]
    table1 = jnp.concatenate([coords, features], axis=-1)
    table1 = jnp.pad(table1, ((0, 0), (0, 0), (0, 10))).reshape(_B * _N, 16)
    g1 = _sc_gather(table1, gidx.reshape(-1))
    cen16 = jnp.pad(centers1, ((0, 0), (0, 0), (0, 13))).reshape(_B * _M1, 16)
    w1a = jnp.pad(W1a, ((0, 10), (0, 0)))
    x1 = _encode(g1, cen16, w1a, b1a.reshape(1, -1), W2a, b2a.reshape(1, -1),
                 W3a, b3a.reshape(1, -1), _K1, 2048)
    x1r = x1.reshape(_B, _M1, 128)

    ct2 = jnp.stack([ox, oy, oz, zm, zm, zm, zm, zm], axis=1)
    kidx2 = _knn(ct2, cen_pad[:, :_M2], _M2, _M1, _K2)
    gidx2 = kidx2 + (jnp.arange(_B, dtype=jnp.int32) * _M1)[:, None, None]
    table2 = jnp.concatenate(
        [centers1, x1r, jnp.zeros((_B, _M1, 13), jnp.float32)], axis=-1
    ).reshape(_B * _M1, 144)
    g2 = _sc_gather(table2, gidx2.reshape(-1))
    cen144 = jnp.pad(centers1[:, :_M2],
                     ((0, 0), (0, 0), (0, 141))).reshape(_B * _M2, 144)
    w1b = jnp.pad(W1b, ((0, 13), (0, 0)))
    x2 = _encode(g2, cen144, w1b, b1b.reshape(1, -1), W2b, b2b.reshape(1, -1),
                 W3b, b3b.reshape(1, -1), _K2, 2048)
    x2r = x2.reshape(_B, _M2, 256)

    return (centers1, x1r, centers1[:, :_M2], x2r)


# speedup vs baseline: 1.0695x; 1.0695x over previous
import functools

import jax
import jax.numpy as jnp
from jax import lax
from jax.experimental import pallas as pl
from jax.experimental.pallas import tpu as pltpu
from jax.experimental.pallas import tpu_sc as plsc

_B, _N = 8, 8192
_M1, _K1 = 512, 32
_M2, _K2 = 128, 16
_NW = 32


def _fps_kernel(x_ref, y_ref, z_ref, ox_ref, oy_ref, oz_ref):
    x = x_ref[...]
    y = y_ref[...]
    z = z_ref[...]
    lane_n = lax.broadcasted_iota(jnp.int32, (_B, _N), 1)
    lane_m = lax.broadcasted_iota(jnp.int32, (_B, _M1), 1)

    cx0 = x[:, 0:1]
    cy0 = y[:, 0:1]
    cz0 = z[:, 0:1]
    ox0 = jnp.where(lane_m == 0, cx0, 0.0)
    oy0 = jnp.where(lane_m == 0, cy0, 0.0)
    oz0 = jnp.where(lane_m == 0, cz0, 0.0)
    dmin0 = jnp.full((_B, _N), jnp.inf, dtype=jnp.float32)

    def body(i, carry):
        cx, cy, cz, ox, oy, oz, dmin = carry
        dx = x - cx
        dy = y - cy
        dz = z - cz
        d = dx * dx + dy * dy + dz * dz
        dmin = jnp.minimum(dmin, d)
        m = jnp.max(dmin, axis=1, keepdims=True)
        cand = jnp.where(dmin == m, lane_n, _N)
        nxt = jnp.min(cand, axis=1, keepdims=True)
        sel = lane_n == nxt
        ncx = jnp.sum(jnp.where(sel, x, 0.0), axis=1, keepdims=True)
        ncy = jnp.sum(jnp.where(sel, y, 0.0), axis=1, keepdims=True)
        ncz = jnp.sum(jnp.where(sel, z, 0.0), axis=1, keepdims=True)
        hit = lane_m == i
        ox = jnp.where(hit, ncx, ox)
        oy = jnp.where(hit, ncy, oy)
        oz = jnp.where(hit, ncz, oz)
        return ncx, ncy, ncz, ox, oy, oz, dmin

    _, _, _, ox, oy, oz, _ = lax.fori_loop(
        1, _M1, body, (cx0, cy0, cz0, ox0, oy0, oz0, dmin0))
    ox_ref[...] = ox
    oy_ref[...] = oy
    oz_ref[...] = oz


def _fps(ct):
    return pl.pallas_call(
        _fps_kernel,
        out_shape=[jax.ShapeDtypeStruct((_B, _M1), jnp.float32)] * 3,
    )(ct[0], ct[1], ct[2])


def _knn_kernel(ct_ref, cen_ref, kidx_ref, d_ref, *, m, n, k):
    ct = ct_ref[0]
    cen = cen_ref[0]
    xx = jnp.sum(ct * ct, axis=0, keepdims=True)
    cc = jnp.sum(cen * cen, axis=1, keepdims=True)
    e = jnp.dot(cen, ct, preferred_element_type=jnp.float32)
    d_ref[...] = cc + xx - 2.0 * e
    lane_n = lax.broadcasted_iota(jnp.int32, (m, n), 1)
    lane_k = lax.broadcasted_iota(jnp.int32, (m, k), 1)

    def step(j, carry):
        prev, kidx = carry
        dm = jnp.where(lane_n == prev, jnp.inf, d_ref[...])
        d_ref[...] = dm
        mn = jnp.min(dm, axis=1, keepdims=True)
        cand = jnp.where(dm == mn, lane_n, n)
        nxt = jnp.min(cand, axis=1, keepdims=True)
        kidx = jnp.where(lane_k == j, nxt, kidx)
        return nxt, kidx

    _, kidx = lax.fori_loop(
        0, k, step,
        (jnp.full((m, 1), -1, jnp.int32), jnp.zeros((m, k), jnp.int32)))
    kidx_ref[0] = kidx


def _knn(ct8, cen, m, n, k):
    return pl.pallas_call(
        functools.partial(_knn_kernel, m=m, n=n, k=k),
        grid=(_B,),
        in_specs=[
            pl.BlockSpec((1, 8, n), lambda b: (b, 0, 0)),
            pl.BlockSpec((1, m, 8), lambda b: (b, 0, 0)),
        ],
        out_specs=pl.BlockSpec((1, m, k), lambda b: (b, 0, 0)),
        out_shape=jax.ShapeDtypeStruct((_B, m, k), jnp.int32),
        scratch_shapes=[pltpu.VMEM((m, n), jnp.float32)],
    )(ct8, cen)


def _sc_gather(table, idx):
    ni = idx.shape[0]
    d = table.shape[1]
    bpw = ni // _NW
    mesh = plsc.VectorSubcoreMesh(core_axis_name="c", subcore_axis_name="s")

    @functools.partial(
        pl.kernel,
        out_type=jax.ShapeDtypeStruct((ni, d), jnp.float32),
        mesh=mesh,
        scratch_types=[
            pltpu.VMEM((bpw,), jnp.int32),
            pltpu.VMEM((bpw, d), jnp.float32),
            pltpu.SemaphoreType.DMA,
        ],
        compiler_params=pltpu.CompilerParams(use_tc_tiling_on_sc=False),
    )
    def gk(table_hbm, idx_hbm, out_hbm, idx_v, rows_v, sem):
        wid = lax.axis_index("s") * 2 + lax.axis_index("c")
        base = wid * bpw
        pltpu.sync_copy(idx_hbm.at[pl.ds(base, bpw)], idx_v)
        pltpu.async_copy(table_hbm.at[idx_v], rows_v, sem).wait()
        pltpu.sync_copy(rows_v, out_hbm.at[pl.ds(base, bpw)])

    return gk(table, idx)


def _enc_kernel(p_ref, c_ref, w1_ref, b1_ref, w2_ref, b2_ref, w3_ref, b3_ref,
                o_ref, *, group):
    g = p_ref[...]
    c = c_ref[...]
    r, kd = g.shape
    cb = jnp.broadcast_to(c[:, None, :], (r // group, group, kd)).reshape(r, kd)
    h = g - cb
    h = jnp.dot(h, w1_ref[...], preferred_element_type=jnp.float32)
    h = jnp.maximum(h + b1_ref[...], 0.0)
    h = jnp.dot(h, w2_ref[...], preferred_element_type=jnp.float32)
    h = jnp.maximum(h + b2_ref[...], 0.0)
    h = jnp.dot(h, w3_ref[...], preferred_element_type=jnp.float32)
    h = jnp.maximum(h + b3_ref[...], 0.0)
    o_ref[...] = jnp.max(h.reshape(r // group, group, -1), axis=1)


def _encode(p, cen, w1, b1, w2, b2, w3, b3, group, tile_rows):
    rows, kdim = p.shape
    grid = rows // tile_rows
    otile = tile_rows // group
    c1, c2, c3 = w1.shape[1], w2.shape[1], w3.shape[1]
    return pl.pallas_call(
        functools.partial(_enc_kernel, group=group),
        grid=(grid,),
        in_specs=[
            pl.BlockSpec((tile_rows, kdim), lambda i: (i, 0)),
            pl.BlockSpec((otile, kdim), lambda i: (i, 0)),
            pl.BlockSpec((kdim, c1), lambda i: (0, 0)),
            pl.BlockSpec((1, c1), lambda i: (0, 0)),
            pl.BlockSpec((c1, c2), lambda i: (0, 0)),
            pl.BlockSpec((1, c2), lambda i: (0, 0)),
            pl.BlockSpec((c2, c3), lambda i: (0, 0)),
            pl.BlockSpec((1, c3), lambda i: (0, 0)),
        ],
        out_specs=pl.BlockSpec((otile, c3), lambda i: (i, 0)),
        out_shape=jax.ShapeDtypeStruct((rows // group, c3), jnp.float32),
    )(p, cen, w1, b1, w2, b2, w3, b3)


def kernel(coords, features, W1a, b1a, W2a, b2a, W3a, b3a,
           W1b, b1b, W2b, b2b, W3b, b3b):
    ct = jnp.transpose(coords, (2, 0, 1))
    ox, oy, oz = _fps(ct)
    centers1 = jnp.stack([ox, oy, oz], axis=-1)
    zm = jnp.zeros((_B, _M1), jnp.float32)
    cen_pad = jnp.stack([ox, oy, oz, zm, zm, zm, zm, zm], axis=-1)
    ct8 = jnp.concatenate(
        [ct, jnp.zeros((5, _B, _N), jnp.float32)], axis=0
    ).transpose(1, 0, 2)

    kidx = _knn(ct8, cen_pad, _M1, _N, _K1)
    gidx = kidx + (jnp.arange(_B, dtype=jnp.int32) * _N)[:, None, None]
    table1 = jnp.concatenate([coords, features], axis=-1)
    table1 = jnp.pad(table1, ((0, 0), (0, 0), (0, 10))).reshape(_B * _N, 16)
    g1 = _sc_gather(table1, gidx.reshape(-1))
    cen16 = jnp.pad(centers1, ((0, 0), (0, 0), (0, 13))).reshape(_B * _M1, 16)
    w1a = jnp.pad(W1a, ((0, 10), (0, 0)))
    x1 = _encode(g1, cen16, w1a, b1a.reshape(1, -1), W2a, b2a.reshape(1, -1),
                 W3a, b3a.reshape(1, -1), _K1, 2048)
    x1r = x1.reshape(_B, _M1, 128)

    ct2 = jnp.stack([ox, oy, oz, zm, zm, zm, zm, zm], axis=1)
    kidx2 = _knn(ct2, cen_pad[:, :_M2], _M2, _M1, _K2)
    gidx2 = kidx2 + (jnp.arange(_B, dtype=jnp.int32) * _M1)[:, None, None]
    table2 = jnp.concatenate(
        [centers1, x1r, jnp.zeros((_B, _M1, 13), jnp.float32)], axis=-1
    ).reshape(_B * _M1, 144)
    g2 = _sc_gather(table2, gidx2.reshape(-1))
    cen144 = jnp.pad(centers1[:, :_M2],
                     ((0, 0), (0, 0), (0, 141))).reshape(_B * _M2, 144)
    w1b = jnp.pad(W1b, ((0, 13), (0, 0)))
    x2 = _encode(g2, cen144, w1b, b1b.reshape(1, -1), W2b, b2b.reshape(1, -1),
                 W3b, b3b.reshape(1, -1), _K2, 2048)
    x2r = x2.reshape(_B, _M2, 256)

    return (centers1, x1r, centers1[:, :_M2], x2r)


# speedup vs baseline: 1.3774x; 1.2879x over previous
import functools

import jax
import jax.numpy as jnp
from jax import lax
from jax.experimental import pallas as pl
from jax.experimental.pallas import tpu as pltpu
from jax.experimental.pallas import tpu_sc as plsc

_B, _N = 8, 8192
_M1, _K1 = 512, 32
_M2, _K2 = 128, 16
_NW = 32


def _fps_kernel(x_ref, y_ref, z_ref, ox_ref, oy_ref, oz_ref):
    x = x_ref[...]
    y = y_ref[...]
    z = z_ref[...]
    lane_n = lax.broadcasted_iota(jnp.int32, (_B, _N), 1)
    lane_m = lax.broadcasted_iota(jnp.int32, (_B, _M1), 1)

    cx0 = x[:, 0:1]
    cy0 = y[:, 0:1]
    cz0 = z[:, 0:1]
    ox0 = jnp.where(lane_m == 0, cx0, 0.0)
    oy0 = jnp.where(lane_m == 0, cy0, 0.0)
    oz0 = jnp.where(lane_m == 0, cz0, 0.0)
    dmin0 = jnp.full((_B, _N), jnp.inf, dtype=jnp.float32)

    def body(i, carry):
        cx, cy, cz, ox, oy, oz, dmin = carry
        dx = x - cx
        dy = y - cy
        dz = z - cz
        d = dx * dx + dy * dy + dz * dz
        dmin = jnp.minimum(dmin, d)
        m = jnp.max(dmin, axis=1, keepdims=True)
        cand = jnp.where(dmin == m, lane_n, _N)
        nxt = jnp.min(cand, axis=1, keepdims=True)
        sel = lane_n == nxt
        ncx = jnp.sum(jnp.where(sel, x, 0.0), axis=1, keepdims=True)
        ncy = jnp.sum(jnp.where(sel, y, 0.0), axis=1, keepdims=True)
        ncz = jnp.sum(jnp.where(sel, z, 0.0), axis=1, keepdims=True)
        hit = lane_m == i
        ox = jnp.where(hit, ncx, ox)
        oy = jnp.where(hit, ncy, oy)
        oz = jnp.where(hit, ncz, oz)
        return ncx, ncy, ncz, ox, oy, oz, dmin

    _, _, _, ox, oy, oz, _ = lax.fori_loop(
        1, _M1, body, (cx0, cy0, cz0, ox0, oy0, oz0, dmin0))
    ox_ref[...] = ox
    oy_ref[...] = oy
    oz_ref[...] = oz


def _fps(ct):
    return pl.pallas_call(
        _fps_kernel,
        out_shape=[jax.ShapeDtypeStruct((_B, _M1), jnp.float32)] * 3,
    )(ct[0], ct[1], ct[2])


def _knn_kernel(ct_ref, cen_ref, kidx_ref, d_ref, *, m, n, k):
    ct = ct_ref[0]
    cen = cen_ref[0]
    xx = jnp.sum(ct * ct, axis=0, keepdims=True)
    cc = jnp.sum(cen * cen, axis=1, keepdims=True)
    e = jnp.dot(cen, ct, preferred_element_type=jnp.float32)
    d_ref[...] = cc + xx - 2.0 * e
    lane_n = lax.broadcasted_iota(jnp.int32, (m, n), 1)
    lane_k = lax.broadcasted_iota(jnp.int32, (m, k), 1)

    def step(j, carry):
        prev, kidx = carry
        dm = jnp.where(lane_n == prev, jnp.inf, d_ref[...])
        d_ref[...] = dm
        mn = jnp.min(dm, axis=1, keepdims=True)
        cand = jnp.where(dm == mn, lane_n, n)
        nxt = jnp.min(cand, axis=1, keepdims=True)
        kidx = jnp.where(lane_k == j, nxt, kidx)
        return nxt, kidx

    _, kidx = lax.fori_loop(
        0, k, step,
        (jnp.full((m, 1), -1, jnp.int32), jnp.zeros((m, k), jnp.int32)))
    kidx_ref[0] = kidx


_NPOP = 48
_NGRP = 1024
_WCAND = 8 * _NPOP


def _knn1a_kernel(ct_ref, cen_ref, kcand_ref, d_ref):
    ct = ct_ref[0]
    cen = cen_ref[0]
    xx = jnp.sum(ct * ct, axis=0, keepdims=True)
    cc = jnp.sum(cen * cen, axis=1, keepdims=True)
    e = jnp.dot(cen, ct, preferred_element_type=jnp.float32)
    d_ref[...] = cc + xx - 2.0 * e
    g = d_ref[:, 0:_NGRP]
    for j in range(1, 8):
        g = jnp.minimum(g, d_ref[:, _NGRP * j:_NGRP * (j + 1)])
    lane_g = lax.broadcasted_iota(jnp.int32, (_M1, _NGRP), 1)
    lane_p = lax.broadcasted_iota(jnp.int32, (_M1, _NPOP), 1)

    def step(t, carry):
        gv, prev, gsel = carry
        gv = jnp.where(lane_g == prev, jnp.inf, gv)
        mn = jnp.min(gv, axis=1, keepdims=True)
        cand = jnp.where(gv == mn, lane_g, _NGRP)
        nxt = jnp.min(cand, axis=1, keepdims=True)
        gsel = jnp.where(lane_p == t, nxt, gsel)
        return gv, nxt, gsel

    _, _, gsel = lax.fori_loop(
        0, _NPOP, step,
        (g, jnp.full((_M1, 1), -1, jnp.int32),
         jnp.zeros((_M1, _NPOP), jnp.int32)))
    tmat = (lax.broadcasted_iota(jnp.int32, (_NPOP, _WCAND), 1) % _NPOP
            == lax.broadcasted_iota(jnp.int32, (_NPOP, _WCAND), 0)
            ).astype(jnp.float32)
    kc = jnp.dot(gsel.astype(jnp.float32), tmat,
                 preferred_element_type=jnp.float32).astype(jnp.int32)
    kc = kc + _NGRP * (
        lax.broadcasted_iota(jnp.int32, (_M1, _WCAND), 1) // _NPOP)
    kcand_ref[0] = kc


def _knn1a(ct8, cen):
    return pl.pallas_call(
        _knn1a_kernel,
        grid=(_B,),
        in_specs=[
            pl.BlockSpec((1, 8, _N), lambda b: (b, 0, 0)),
            pl.BlockSpec((1, _M1, 8), lambda b: (b, 0, 0)),
        ],
        out_specs=pl.BlockSpec((1, _M1, _WCAND), lambda b: (b, 0, 0)),
        out_shape=jax.ShapeDtypeStruct((_B, _M1, _WCAND), jnp.int32),
        scratch_shapes=[pltpu.VMEM((_M1, _N), jnp.float32)],
    )(ct8, cen)


def _knn1b_kernel(xg_ref, yg_ref, zg_ref, kc_ref, cen_ref, kidx_ref):
    xg = xg_ref[0]
    yg = yg_ref[0]
    zg = zg_ref[0]
    kc = kc_ref[0]
    cen = cen_ref[0]
    cx = cen[:, 0:1]
    cy = cen[:, 1:2]
    cz = cen[:, 2:3]
    cc = jnp.sum(cen * cen, axis=1, keepdims=True)
    xx = xg * xg + yg * yg + zg * zg
    ee = xg * cx + yg * cy + zg * cz
    d0 = cc + xx - 2.0 * ee
    lane_k = lax.broadcasted_iota(jnp.int32, (_M1, _K1), 1)

    def step(j, carry):
        dv, prev, kidx = carry
        dv = jnp.where(kc == prev, jnp.inf, dv)
        mn = jnp.min(dv, axis=1, keepdims=True)
        cand = jnp.where(dv == mn, kc, _N)
        nxt = jnp.min(cand, axis=1, keepdims=True)
        kidx = jnp.where(lane_k == j, nxt, kidx)
        return dv, nxt, kidx

    _, _, kidx = lax.fori_loop(
        0, _K1, step,
        (d0, jnp.full((_M1, 1), -1, jnp.int32),
         jnp.zeros((_M1, _K1), jnp.int32)))
    kidx_ref[0] = kidx


def _knn1b(xg, yg, zg, kcand, cen):
    return pl.pallas_call(
        _knn1b_kernel,
        grid=(_B,),
        in_specs=[
            pl.BlockSpec((1, _M1, _WCAND), lambda b: (b, 0, 0)),
            pl.BlockSpec((1, _M1, _WCAND), lambda b: (b, 0, 0)),
            pl.BlockSpec((1, _M1, _WCAND), lambda b: (b, 0, 0)),
            pl.BlockSpec((1, _M1, _WCAND), lambda b: (b, 0, 0)),
            pl.BlockSpec((1, _M1, 8), lambda b: (b, 0, 0)),
        ],
        out_specs=pl.BlockSpec((1, _M1, _K1), lambda b: (b, 0, 0)),
        out_shape=jax.ShapeDtypeStruct((_B, _M1, _K1), jnp.int32),
    )(xg, yg, zg, kcand, cen)


def _sc_gather_planes(xt, yt, zt, idx):
    ni = idx.shape[0]
    bpw = ni // _NW
    mesh = plsc.VectorSubcoreMesh(core_axis_name="c", subcore_axis_name="s")
    out_t = jax.ShapeDtypeStruct((ni,), jnp.float32)

    @functools.partial(
        pl.kernel,
        out_type=(out_t, out_t, out_t),
        mesh=mesh,
        scratch_types=[
            pltpu.VMEM((bpw,), jnp.int32),
            pltpu.VMEM((bpw,), jnp.float32),
            pltpu.SemaphoreType.DMA,
        ],
        compiler_params=pltpu.CompilerParams(use_tc_tiling_on_sc=False),
    )
    def gk(xh, yh, zh, idx_hbm, oxh, oyh, ozh, idx_v, rows_v, sem):
        wid = lax.axis_index("s") * 2 + lax.axis_index("c")
        base = wid * bpw
        pltpu.sync_copy(idx_hbm.at[pl.ds(base, bpw)], idx_v)
        for src, dst in ((xh, oxh), (yh, oyh), (zh, ozh)):
            pltpu.async_copy(src.at[idx_v], rows_v, sem).wait()
            pltpu.sync_copy(rows_v, dst.at[pl.ds(base, bpw)])

    return gk(xt, yt, zt, idx)


def _knn(ct8, cen, m, n, k):
    return pl.pallas_call(
        functools.partial(_knn_kernel, m=m, n=n, k=k),
        grid=(_B,),
        in_specs=[
            pl.BlockSpec((1, 8, n), lambda b: (b, 0, 0)),
            pl.BlockSpec((1, m, 8), lambda b: (b, 0, 0)),
        ],
        out_specs=pl.BlockSpec((1, m, k), lambda b: (b, 0, 0)),
        out_shape=jax.ShapeDtypeStruct((_B, m, k), jnp.int32),
        scratch_shapes=[pltpu.VMEM((m, n), jnp.float32)],
    )(ct8, cen)


def _sc_gather(table, idx):
    ni = idx.shape[0]
    d = table.shape[1]
    bpw = ni // _NW
    mesh = plsc.VectorSubcoreMesh(core_axis_name="c", subcore_axis_name="s")

    @functools.partial(
        pl.kernel,
        out_type=jax.ShapeDtypeStruct((ni, d), jnp.float32),
        mesh=mesh,
        scratch_types=[
            pltpu.VMEM((bpw,), jnp.int32),
            pltpu.VMEM((bpw, d), jnp.float32),
            pltpu.SemaphoreType.DMA,
        ],
        compiler_params=pltpu.CompilerParams(use_tc_tiling_on_sc=False),
    )
    def gk(table_hbm, idx_hbm, out_hbm, idx_v, rows_v, sem):
        wid = lax.axis_index("s") * 2 + lax.axis_index("c")
        base = wid * bpw
        pltpu.sync_copy(idx_hbm.at[pl.ds(base, bpw)], idx_v)
        pltpu.async_copy(table_hbm.at[idx_v], rows_v, sem).wait()
        pltpu.sync_copy(rows_v, out_hbm.at[pl.ds(base, bpw)])

    return gk(table, idx)


def _enc_kernel(p_ref, c_ref, w1_ref, b1_ref, w2_ref, b2_ref, w3_ref, b3_ref,
                o_ref, *, group):
    g = p_ref[...]
    c = c_ref[...]
    r, kd = g.shape
    cb = jnp.broadcast_to(c[:, None, :], (r // group, group, kd)).reshape(r, kd)
    h = g - cb
    h = jnp.dot(h, w1_ref[...], preferred_element_type=jnp.float32)
    h = jnp.maximum(h + b1_ref[...], 0.0)
    h = jnp.dot(h, w2_ref[...], preferred_element_type=jnp.float32)
    h = jnp.maximum(h + b2_ref[...], 0.0)
    h = jnp.dot(h, w3_ref[...], preferred_element_type=jnp.float32)
    h = jnp.maximum(h + b3_ref[...], 0.0)
    o_ref[...] = jnp.max(h.reshape(r // group, group, -1), axis=1)


def _encode(p, cen, w1, b1, w2, b2, w3, b3, group, tile_rows):
    rows, kdim = p.shape
    grid = rows // tile_rows
    otile = tile_rows // group
    c1, c2, c3 = w1.shape[1], w2.shape[1], w3.shape[1]
    return pl.pallas_call(
        functools.partial(_enc_kernel, group=group),
        grid=(grid,),
        in_specs=[
            pl.BlockSpec((tile_rows, kdim), lambda i: (i, 0)),
            pl.BlockSpec((otile, kdim), lambda i: (i, 0)),
            pl.BlockSpec((kdim, c1), lambda i: (0, 0)),
            pl.BlockSpec((1, c1), lambda i: (0, 0)),
            pl.BlockSpec((c1, c2), lambda i: (0, 0)),
            pl.BlockSpec((1, c2), lambda i: (0, 0)),
            pl.BlockSpec((c2, c3), lambda i: (0, 0)),
            pl.BlockSpec((1, c3), lambda i: (0, 0)),
        ],
        out_specs=pl.BlockSpec((otile, c3), lambda i: (i, 0)),
        out_shape=jax.ShapeDtypeStruct((rows // group, c3), jnp.float32),
    )(p, cen, w1, b1, w2, b2, w3, b3)


def kernel(coords, features, W1a, b1a, W2a, b2a, W3a, b3a,
           W1b, b1b, W2b, b2b, W3b, b3b):
    ct = jnp.transpose(coords, (2, 0, 1))
    ox, oy, oz = _fps(ct)
    centers1 = jnp.stack([ox, oy, oz], axis=-1)
    zm = jnp.zeros((_B, _M1), jnp.float32)
    cen_pad = jnp.stack([ox, oy, oz, zm, zm, zm, zm, zm], axis=-1)
    ct8 = jnp.concatenate(
        [ct, jnp.zeros((5, _B, _N), jnp.float32)], axis=0
    ).transpose(1, 0, 2)

    kcand = _knn1a(ct8, cen_pad)
    gcand = kcand + (jnp.arange(_B, dtype=jnp.int32) * _N)[:, None, None]
    xg, yg, zg = _sc_gather_planes(
        ct[0].reshape(-1), ct[1].reshape(-1), ct[2].reshape(-1),
        gcand.reshape(-1))
    kidx = _knn1b(xg.reshape(_B, _M1, _WCAND), yg.reshape(_B, _M1, _WCAND),
                  zg.reshape(_B, _M1, _WCAND), kcand, cen_pad)
    gidx = kidx + (jnp.arange(_B, dtype=jnp.int32) * _N)[:, None, None]
    table1 = jnp.concatenate([coords, features], axis=-1)
    table1 = jnp.pad(table1, ((0, 0), (0, 0), (0, 10))).reshape(_B * _N, 16)
    g1 = _sc_gather(table1, gidx.reshape(-1))
    cen16 = jnp.pad(centers1, ((0, 0), (0, 0), (0, 13))).reshape(_B * _M1, 16)
    w1a = jnp.pad(W1a, ((0, 10), (0, 0)))
    x1 = _encode(g1, cen16, w1a, b1a.reshape(1, -1), W2a, b2a.reshape(1, -1),
                 W3a, b3a.reshape(1, -1), _K1, 2048)
    x1r = x1.reshape(_B, _M1, 128)

    ct2 = jnp.stack([ox, oy, oz, zm, zm, zm, zm, zm], axis=1)
    kidx2 = _knn(ct2, cen_pad[:, :_M2], _M2, _M1, _K2)
    gidx2 = kidx2 + (jnp.arange(_B, dtype=jnp.int32) * _M1)[:, None, None]
    table2 = jnp.concatenate(
        [centers1, x1r, jnp.zeros((_B, _M1, 13), jnp.float32)], axis=-1
    ).reshape(_B * _M1, 144)
    g2 = _sc_gather(table2, gidx2.reshape(-1))
    cen144 = jnp.pad(centers1[:, :_M2],
                     ((0, 0), (0, 0), (0, 141))).reshape(_B * _M2, 144)
    w1b = jnp.pad(W1b, ((0, 13), (0, 0)))
    x2 = _encode(g2, cen144, w1b, b1b.reshape(1, -1), W2b, b2b.reshape(1, -1),
                 W3b, b3b.reshape(1, -1), _K2, 2048)
    x2r = x2.reshape(_B, _M2, 256)

    return (centers1, x1r, centers1[:, :_M2], x2r)
